# baseline (speedup 1.0000x reference)
import jax
import jax.numpy as jnp
from jax.experimental import pallas as pl

N, D, H = 10000, 128, 128
Hh = H // 2
BR1 = 400
BRM = 1000
GRID1 = N // BR1
GRIDM = N // BRM
BF16 = jnp.bfloat16
F32 = jnp.float32


def _bdot(a, b):
    return jnp.dot(a, b, preferred_element_type=F32)


def _gconv(a_bf, hb_ref, hf_ref, wt_ref, wb_ref, b_ref):
    agg = _bdot(a_bf, hf_ref[...])
    h = _bdot(hb_ref[...], wt_ref[...])
    h = h + _bdot(agg.astype(BF16), wb_ref[...])
    return jnp.maximum(h + b_ref[...], 0.0)


def _layer1_body(adj_ref, xb_ref, xf_ref, wt_ref, wb_ref, b_ref,
                 abf_ref, h_ref):
    a_bf = adj_ref[...].astype(BF16)
    abf_ref[...] = a_bf
    h = _gconv(a_bf, xb_ref, xf_ref, wt_ref, wb_ref, b_ref)
    h_ref[...] = h.astype(BF16)


def _mid_body(abf_ref, hb_ref, hf_ref, wt_ref, wb_ref, b_ref, h_ref):
    h = _gconv(abf_ref[...], hb_ref, hf_ref, wt_ref, wb_ref, b_ref)
    h_ref[...] = h.astype(BF16)


def _last_body(abf_ref, hb_ref, hf_ref, wt_ref, wb_ref, b_ref,
               cw1_ref, cb1_ref, pa_ref, cw2_ref, cb2_ref, out_ref):
    h = _gconv(abf_ref[...], hb_ref, hf_ref, wt_ref, wb_ref, b_ref)
    z = _bdot(h.astype(BF16), cw1_ref[...]) + cb1_ref[...]
    z = jnp.where(z >= 0, z, pa_ref[...] * z)
    out_ref[...] = _bdot(z.astype(BF16), cw2_ref[...]) + cb2_ref[...]


def _full(shape):
    return pl.BlockSpec(shape, lambda i: tuple(0 for _ in shape))


def _rowblk(br, cols):
    return pl.BlockSpec((br, cols), lambda i: (i, 0))


@jax.jit
def kernel(x, adj, W1, b1, W2, b2, W3, b3, W4, b4, cW1, cb1, pa, cW2, cb2):
    xf = x.astype(BF16)

    abf, h1 = pl.pallas_call(
        _layer1_body,
        grid=(GRID1,),
        in_specs=[_rowblk(BR1, N), _rowblk(BR1, D), _full((N, D)),
                  _full((D, H)), _full((D, H)), _full((1, H))],
        out_specs=[_rowblk(BR1, N), _rowblk(BR1, H)],
        out_shape=[jax.ShapeDtypeStruct((N, N), BF16),
                   jax.ShapeDtypeStruct((N, H), BF16)],
    )(adj, xf, xf, W1[:D].astype(BF16), W1[D:].astype(BF16),
      b1.reshape(1, H))

    def mid(h_prev, W, b, dim_in, dim_out):
        return pl.pallas_call(
            _mid_body,
            grid=(GRIDM,),
            in_specs=[_rowblk(BRM, N), _rowblk(BRM, dim_in),
                      _full((N, dim_in)), _full((dim_in, dim_out)),
                      _full((dim_in, dim_out)), _full((1, dim_out))],
            out_specs=_rowblk(BRM, dim_out),
            out_shape=jax.ShapeDtypeStruct((N, dim_out), BF16),
        )(abf, h_prev, h_prev, W[:dim_in].astype(BF16),
          W[dim_in:].astype(BF16), b.reshape(1, dim_out))

    h2 = mid(h1, W2, b2, H, H)
    h3 = mid(h2, W3, b3, H, Hh)

    pred = pl.pallas_call(
        _last_body,
        grid=(GRIDM,),
        in_specs=[_rowblk(BRM, N), _rowblk(BRM, Hh), _full((N, Hh)),
                  _full((Hh, Hh)), _full((Hh, Hh)), _full((1, Hh)),
                  _full((Hh, Hh)), _full((1, Hh)), _full((1, Hh)),
                  _full((Hh, 2)), _full((1, 2))],
        out_specs=_rowblk(BRM, 2),
        out_shape=jax.ShapeDtypeStruct((N, 2), F32),
    )(abf, h3, h3, W4[:Hh].astype(BF16), W4[Hh:].astype(BF16),
      b4.reshape(1, Hh), cW1.astype(BF16), cb1.reshape(1, Hh),
      pa.reshape(1, Hh), cW2.astype(BF16), cb2.reshape(1, 2))

    return pred
